# baseline (speedup 1.0000x reference)
import jax
import jax.numpy as jnp
from jax.experimental import pallas as pl
from jax.experimental.pallas import tpu as pltpu

_N = 4096
_CE = 4
_CN = 128
_COUT = 128

_BI = 128
_NI = _N // _BI
_NBUF = 3
_NQ = 4
_HQ = 16 // _NQ


def _body(a_hbm, wp_ref, x_ref, wself_ref, b_ref, dinv_ref, o_ref, abuf, sems):
    i = pl.program_id(0)

    def start(blk, buf):
        for q in range(_NQ):
            pltpu.make_async_copy(
                a_hbm.at[pl.ds(blk * _BI, _BI), pl.ds(q * _HQ, _HQ)],
                abuf.at[buf, :, pl.ds(q * _HQ, _HQ)],
                sems.at[buf, q],
            ).start()

    @pl.when(i == 0)
    def _prime():
        for k in range(_NBUF):
            start(k, k)

    buf = jax.lax.rem(i, _NBUF)
    for q in range(_NQ):
        pltpu.make_async_copy(
            a_hbm.at[pl.ds(i * _BI, _BI), pl.ds(q * _HQ, _HQ)],
            abuf.at[buf, :, pl.ds(q * _HQ, _HQ)],
            sems.at[buf, q],
        ).wait()

    x = abuf[buf, :, 0]
    for t in range(1, 16):
        x = x + abuf[buf, :, t]
    x = x[:, :4, :] + x[:, 4:, :]
    acc = jnp.sum(x, axis=2)

    nxt = i + _NBUF

    @pl.when(nxt < _NI)
    def _refill():
        start(nxt, buf)

    msg = (
        jnp.dot(acc, wp_ref[...], preferred_element_type=jnp.float32)
        * dinv_ref[...]
    )
    self_t = jnp.dot(
        x_ref[...], wself_ref[...], preferred_element_type=jnp.float32
    )
    o_ref[...] = jnp.maximum(msg + self_t + b_ref[...], 0.0)


def kernel(D, A, X, W_pass, b_pass, W_self, b_self):
    A4 = (
        A.reshape(_N, 16, 2, 128, _CE)
        .transpose(0, 1, 2, 4, 3)
        .reshape(_N, 16, 8, 128)
    )
    Wp_T = W_pass.T
    Wself_T = W_self.T
    b = (b_pass + b_self).reshape(1, _COUT)
    Dinv = (1.0 / D).reshape(_N, 1)

    out = pl.pallas_call(
        _body,
        grid=(_NI,),
        in_specs=[
            pl.BlockSpec(memory_space=pltpu.MemorySpace.HBM),
            pl.BlockSpec((_CE, _COUT), lambda i: (0, 0)),
            pl.BlockSpec((_BI, _CN), lambda i: (i, 0)),
            pl.BlockSpec((_CN, _COUT), lambda i: (0, 0)),
            pl.BlockSpec((1, _COUT), lambda i: (0, 0)),
            pl.BlockSpec((_BI, 1), lambda i: (i, 0)),
        ],
        out_specs=pl.BlockSpec((_BI, _COUT), lambda i: (i, 0)),
        out_shape=jax.ShapeDtypeStruct((_N, _COUT), jnp.float32),
        scratch_shapes=[
            pltpu.VMEM((_NBUF, _BI, 16, 8, 128), jnp.float32),
            pltpu.SemaphoreType.DMA((_NBUF, _NQ)),
        ],
        compiler_params=pltpu.CompilerParams(
            dimension_semantics=("arbitrary",),
        ),
    )(A4, Wp_T, X, Wself_T, b, Dinv)
    return out


# speedup vs baseline: 1.0183x; 1.0183x over previous
import jax
import jax.numpy as jnp
from jax.experimental import pallas as pl
from jax.experimental.pallas import tpu as pltpu

_N = 4096
_CE = 4
_CN = 128
_COUT = 128

_BI = 128
_NI = _N // _BI
_NW = 8
_HW = 16 // _NW


def _body(*refs):
    a_refs = refs[:_NW]
    wp_ref, x_ref, wself_ref, b_ref, dinv_ref, o_ref = refs[_NW:]

    x = a_refs[0][:, 0]
    for w in range(_NW):
        for t in range(_HW):
            if w == 0 and t == 0:
                continue
            x = x + a_refs[w][:, t]
    x = x[:, :4, :] + x[:, 4:, :]
    acc = jnp.sum(x, axis=2)
    msg = (
        jnp.dot(acc, wp_ref[...], preferred_element_type=jnp.float32)
        * dinv_ref[...]
    )
    self_t = jnp.dot(
        x_ref[...], wself_ref[...], preferred_element_type=jnp.float32
    )
    o_ref[...] = jnp.maximum(msg + self_t + b_ref[...], 0.0)


def kernel(D, A, X, W_pass, b_pass, W_self, b_self):
    A4 = (
        A.reshape(_N, 16, 2, 128, _CE)
        .transpose(0, 1, 2, 4, 3)
        .reshape(_N, 16, 8, 128)
    )
    Wp_T = W_pass.T
    Wself_T = W_self.T
    b = (b_pass + b_self).reshape(1, _COUT)
    Dinv = (1.0 / D).reshape(_N, 1)

    a_specs = [
        pl.BlockSpec((_BI, _HW, 8, 128), lambda i, w=w: (i, w, 0, 0))
        for w in range(_NW)
    ]
    out = pl.pallas_call(
        _body,
        grid=(_NI,),
        in_specs=a_specs
        + [
            pl.BlockSpec((_CE, _COUT), lambda i: (0, 0)),
            pl.BlockSpec((_BI, _CN), lambda i: (i, 0)),
            pl.BlockSpec((_CN, _COUT), lambda i: (0, 0)),
            pl.BlockSpec((1, _COUT), lambda i: (0, 0)),
            pl.BlockSpec((_BI, 1), lambda i: (i, 0)),
        ],
        out_specs=pl.BlockSpec((_BI, _COUT), lambda i: (i, 0)),
        out_shape=jax.ShapeDtypeStruct((_N, _COUT), jnp.float32),
        compiler_params=pltpu.CompilerParams(
            dimension_semantics=("arbitrary",),
        ),
    )(*([A4] * _NW), Wp_T, X, Wself_T, b, Dinv)
    return out
